# baseline (speedup 1.0000x reference)
import functools

import jax
import jax.numpy as jnp
from jax import lax
from jax.experimental import pallas as pl
from jax.experimental.pallas import tpu as pltpu
from jax.experimental.pallas import tpu_sc as plsc

BATCH = 4096
IN_DIM = 768
E_DIM = 32
K = 8192
BETA = 0.25
TILE = 512

_dot = functools.partial(jax.lax.dot_general, preferred_element_type=jnp.float32)
_NC, _NS = 2, 16
_NW = _NC * _NS


def _argmin_idx(r, cbT):
    csq = jnp.sum(cbT * cbT, axis=0, keepdims=True)
    s = csq + _dot(-2.0 * r, cbT, (((1,), (0,)), ((), ())))
    m = jnp.min(s, axis=1, keepdims=True)
    lane = jax.lax.broadcasted_iota(jnp.int32, s.shape, 1)
    return jnp.min(jnp.where(s == m, lane, K), axis=1, keepdims=True)


def _encode_kernel(x_ref, We0_ref, be0_ref, We1_ref, be1_ref, We2_ref, be2_ref,
                   cbT_ref, z_ref, idx_ref):
    h = jnp.maximum(_dot(x_ref[...], We0_ref[...], (((1,), (0,)), ((), ()))) + be0_ref[...], 0.0)
    h = jnp.maximum(_dot(h, We1_ref[...], (((1,), (0,)), ((), ()))) + be1_ref[...], 0.0)
    z = _dot(h, We2_ref[...], (((1,), (0,)), ((), ()))) + be2_ref[...]
    z_ref[...] = z
    idx_ref[...] = _argmin_idx(z, cbT_ref[...])


def _step_kernel(r_ref, q_ref, acc_ref, cbT_ref, rout_ref, accout_ref, idx_ref,
                 lsum_ref):
    @pl.when(pl.program_id(0) == 0)
    def _init():
        lsum_ref[...] = jnp.zeros_like(lsum_ref)

    r = r_ref[...]
    t = q_ref[...] - r
    lsum_ref[...] += jnp.sum(t * t)
    q_st = r + t
    accout_ref[...] = acc_ref[...] + q_st
    rnew = r - q_st
    rout_ref[...] = rnew
    idx_ref[...] = _argmin_idx(rnew, cbT_ref[...])


def _decode_kernel(r_ref, q_ref, acc_ref, Wd0_ref, bd0_ref, Wd1_ref, bd1_ref,
                   Wd2_ref, bd2_ref, out_ref, lsum_ref):
    @pl.when(pl.program_id(0) == 0)
    def _init():
        lsum_ref[...] = jnp.zeros_like(lsum_ref)

    r = r_ref[...]
    t = q_ref[...] - r
    lsum_ref[...] += jnp.sum(t * t)
    xq = acc_ref[...] + (r + t)
    h = jnp.maximum(_dot(xq, Wd0_ref[...], (((1,), (0,)), ((), ()))) + bd0_ref[...], 0.0)
    h = jnp.maximum(_dot(h, Wd1_ref[...], (((1,), (0,)), ((), ()))) + bd1_ref[...], 0.0)
    out_ref[...] = _dot(h, Wd2_ref[...], (((1,), (0,)), ((), ()))) + bd2_ref[...]


NCHAINS = 4
HALF = BATCH // NCHAINS


def _row_block(d1):
    return pl.BlockSpec((TILE, d1), lambda i: (i, 0))


def _full(shape):
    return pl.BlockSpec(shape, lambda i: (0,) * len(shape))


_GRID = (HALF // TILE,)
_IDX_SPEC = pl.BlockSpec((TILE, 1), lambda i: (i, 0))
_IDX_TYPE = jax.ShapeDtypeStruct((HALF, 1), jnp.int32)
_LSUM_SPEC = _full((1, 128))
_LSUM_TYPE = jax.ShapeDtypeStruct((1, 128), jnp.float32)
_VEC_SPEC = pl.BlockSpec((TILE, E_DIM), lambda i: (i, 0))
_VEC_TYPE = jax.ShapeDtypeStruct((HALF, E_DIM), jnp.float32)


def _sc_gather(table, idx):
    nrows = idx.shape[0]
    bpw = nrows // _NW
    width = table.shape[1]
    mesh = plsc.VectorSubcoreMesh(core_axis_name="c", subcore_axis_name="s")

    @functools.partial(
        pl.kernel,
        out_type=jax.ShapeDtypeStruct((nrows, width), jnp.float32),
        mesh=mesh,
        scratch_types=[
            pltpu.VMEM((bpw,), jnp.int32),
            pltpu.VMEM((bpw, width), jnp.float32),
            pltpu.SemaphoreType.DMA,
        ],
    )
    def gather(table_hbm, idx_hbm, out_hbm, idx_v, rows_v, sem):
        wid = lax.axis_index("s") * _NC + lax.axis_index("c")
        base = wid * bpw
        pltpu.sync_copy(idx_hbm.at[pl.ds(base, bpw)], idx_v)
        pltpu.async_copy(table_hbm.at[idx_v], rows_v, sem).wait()
        pltpu.sync_copy(rows_v, out_hbm.at[pl.ds(base, bpw)])

    return gather(table, idx)


def kernel(x, We0, be0, We1, be1, We2, be2, Wd0, bd0, Wd1, bd1, Wd2, bd2, codebooks):
    cbT = jnp.transpose(codebooks, (0, 2, 1))
    cb_pad = jnp.pad(codebooks, ((0, 0), (0, 0), (0, 128 - E_DIM)))

    encode = pl.pallas_call(
        _encode_kernel,
        grid=_GRID,
        in_specs=[
            _row_block(IN_DIM),
            _full((IN_DIM, 512)), _full((1, 512)),
            _full((512, 128)), _full((1, 128)),
            _full((128, E_DIM)), _full((1, E_DIM)),
            _full((E_DIM, K)),
        ],
        out_specs=[_VEC_SPEC, _IDX_SPEC],
        out_shape=[_VEC_TYPE, _IDX_TYPE],
    )
    step = pl.pallas_call(
        _step_kernel,
        grid=_GRID,
        in_specs=[_VEC_SPEC, _VEC_SPEC, _VEC_SPEC, _full((E_DIM, K))],
        out_specs=[_VEC_SPEC, _VEC_SPEC, _IDX_SPEC, _LSUM_SPEC],
        out_shape=[_VEC_TYPE, _VEC_TYPE, _IDX_TYPE, _LSUM_TYPE],
    )
    decode = pl.pallas_call(
        _decode_kernel,
        grid=_GRID,
        in_specs=[
            _VEC_SPEC, _VEC_SPEC, _VEC_SPEC,
            _full((E_DIM, 128)), _full((1, 128)),
            _full((128, 512)), _full((1, 512)),
            _full((512, IN_DIM)), _full((1, IN_DIM)),
        ],
        out_specs=[_row_block(IN_DIM), _LSUM_SPEC],
        out_shape=[jax.ShapeDtypeStruct((HALF, IN_DIM), jnp.float32), _LSUM_TYPE],
    )

    be = [b.reshape(1, -1) for b in (be0, be1, be2)]
    bd = [b.reshape(1, -1) for b in (bd0, bd1, bd2)]
    halves = []
    for h in range(NCHAINS):
        xh = x[h * HALF:(h + 1) * HALF]
        z, idx = encode(xh, We0, be[0], We1, be[1], We2, be[2], cbT[0])
        halves.append({"r": z, "acc": jnp.zeros((HALF, E_DIM), jnp.float32),
                       "idxs": [idx], "lsums": []})
    for l in range(3):
        for h in halves:
            q = _sc_gather(cb_pad[l], h["idxs"][-1].reshape(-1))[:, :E_DIM]
            h["r"], h["acc"], idx, ls = step(h["r"], q, h["acc"], cbT[l + 1])
            h["idxs"].append(idx)
            h["lsums"].append(ls)
    outs = []
    for h in halves:
        q3 = _sc_gather(cb_pad[3], h["idxs"][-1].reshape(-1))[:, :E_DIM]
        out_h, ls3 = decode(h["r"], q3, h["acc"], Wd0, bd[0], Wd1, bd[1], Wd2, bd[2])
        h["lsums"].append(ls3)
        outs.append(out_h)

    per_level = jnp.stack(
        [sum(h["lsums"][l][0, 0] for h in halves) for l in range(4)]
    ) / (BATCH * E_DIM)
    rq_loss = jnp.mean((1.0 + BETA) * per_level)
    orth_loss = jnp.zeros((), dtype=jnp.float32)
    indices = jnp.concatenate(
        [jnp.concatenate(h["idxs"], axis=1) for h in halves], axis=0)
    out = jnp.concatenate(outs, axis=0)
    return (out, rq_loss, orth_loss, indices)


# speedup vs baseline: 1.1253x; 1.1253x over previous
import functools

import jax
import jax.numpy as jnp
from jax import lax
from jax.experimental import pallas as pl
from jax.experimental.pallas import tpu as pltpu
from jax.experimental.pallas import tpu_sc as plsc

BATCH = 4096
IN_DIM = 768
E_DIM = 32
K = 8192
BETA = 0.25
TILE = 512

_dot = functools.partial(jax.lax.dot_general, preferred_element_type=jnp.float32)
_NC, _NS = 2, 16
_NW = _NC * _NS


def _argmin_idx(r, cbT):
    csq = jnp.sum(cbT * cbT, axis=0, keepdims=True)
    s = csq + _dot(-2.0 * r, cbT, (((1,), (0,)), ((), ())))
    m = jnp.min(s, axis=1, keepdims=True)
    lane = jax.lax.broadcasted_iota(jnp.int32, s.shape, 1)
    return jnp.min(jnp.where(s == m, lane, K), axis=1, keepdims=True)


def _encode_kernel(x_ref, We0_ref, be0_ref, We1_ref, be1_ref, We2_ref, be2_ref,
                   cbT_ref, z_ref, idx_ref):
    h = jnp.maximum(_dot(x_ref[...], We0_ref[...], (((1,), (0,)), ((), ()))) + be0_ref[...], 0.0)
    h = jnp.maximum(_dot(h, We1_ref[...], (((1,), (0,)), ((), ()))) + be1_ref[...], 0.0)
    z = _dot(h, We2_ref[...], (((1,), (0,)), ((), ()))) + be2_ref[...]
    z_ref[...] = z
    idx_ref[...] = _argmin_idx(z, cbT_ref[...])


def _step_kernel(r_ref, q_ref, acc_ref, cbT_ref, rout_ref, accout_ref, idx_ref,
                 lsum_ref):
    @pl.when(pl.program_id(0) == 0)
    def _init():
        lsum_ref[...] = jnp.zeros_like(lsum_ref)

    r = r_ref[...]
    t = q_ref[...] - r
    lsum_ref[...] += jnp.sum(t * t)
    q_st = r + t
    accout_ref[...] = acc_ref[...] + q_st
    rnew = r - q_st
    rout_ref[...] = rnew
    idx_ref[...] = _argmin_idx(rnew, cbT_ref[...])


def _decode_kernel(r_ref, q_ref, acc_ref, Wd0_ref, bd0_ref, Wd1_ref, bd1_ref,
                   Wd2_ref, bd2_ref, out_ref, lsum_ref):
    @pl.when(pl.program_id(0) == 0)
    def _init():
        lsum_ref[...] = jnp.zeros_like(lsum_ref)

    r = r_ref[...]
    t = q_ref[...] - r
    lsum_ref[...] += jnp.sum(t * t)
    xq = acc_ref[...] + (r + t)
    h = jnp.maximum(_dot(xq, Wd0_ref[...], (((1,), (0,)), ((), ()))) + bd0_ref[...], 0.0)
    h = jnp.maximum(_dot(h, Wd1_ref[...], (((1,), (0,)), ((), ()))) + bd1_ref[...], 0.0)
    out_ref[...] = _dot(h, Wd2_ref[...], (((1,), (0,)), ((), ()))) + bd2_ref[...]


NCHAINS = 2
HALF = BATCH // NCHAINS


def _row_block(d1):
    return pl.BlockSpec((TILE, d1), lambda i: (i, 0))


def _full(shape):
    return pl.BlockSpec(shape, lambda i: (0,) * len(shape))


_GRID = (HALF // TILE,)
_IDX_SPEC = pl.BlockSpec((TILE, 1), lambda i: (i, 0))
_IDX_TYPE = jax.ShapeDtypeStruct((HALF, 1), jnp.int32)
_LSUM_SPEC = _full((1, 128))
_LSUM_TYPE = jax.ShapeDtypeStruct((1, 128), jnp.float32)
_VEC_SPEC = pl.BlockSpec((TILE, E_DIM), lambda i: (i, 0))
_VEC_TYPE = jax.ShapeDtypeStruct((HALF, E_DIM), jnp.float32)


def _sc_gather(table, idx):
    nrows = idx.shape[0]
    bpw = nrows // _NW
    width = table.shape[1]
    mesh = plsc.VectorSubcoreMesh(core_axis_name="c", subcore_axis_name="s")

    @functools.partial(
        pl.kernel,
        out_type=jax.ShapeDtypeStruct((nrows, width), jnp.float32),
        mesh=mesh,
        scratch_types=[
            pltpu.VMEM((bpw,), jnp.int32),
            pltpu.VMEM((bpw, width), jnp.float32),
            pltpu.SemaphoreType.DMA,
        ],
    )
    def gather(table_hbm, idx_hbm, out_hbm, idx_v, rows_v, sem):
        wid = lax.axis_index("s") * _NC + lax.axis_index("c")
        base = wid * bpw
        pltpu.sync_copy(idx_hbm.at[pl.ds(base, bpw)], idx_v)
        pltpu.async_copy(table_hbm.at[idx_v], rows_v, sem).wait()
        pltpu.sync_copy(rows_v, out_hbm.at[pl.ds(base, bpw)])

    return gather(table, idx)


def kernel(x, We0, be0, We1, be1, We2, be2, Wd0, bd0, Wd1, bd1, Wd2, bd2, codebooks):
    cbT = jnp.transpose(codebooks, (0, 2, 1))
    cb_pad = jnp.pad(codebooks, ((0, 0), (0, 0), (0, 128 - E_DIM)))

    encode = pl.pallas_call(
        _encode_kernel,
        grid=_GRID,
        in_specs=[
            _row_block(IN_DIM),
            _full((IN_DIM, 512)), _full((1, 512)),
            _full((512, 128)), _full((1, 128)),
            _full((128, E_DIM)), _full((1, E_DIM)),
            _full((E_DIM, K)),
        ],
        out_specs=[_VEC_SPEC, _IDX_SPEC],
        out_shape=[_VEC_TYPE, _IDX_TYPE],
    )
    step = pl.pallas_call(
        _step_kernel,
        grid=_GRID,
        in_specs=[_VEC_SPEC, _VEC_SPEC, _VEC_SPEC, _full((E_DIM, K))],
        out_specs=[_VEC_SPEC, _VEC_SPEC, _IDX_SPEC, _LSUM_SPEC],
        out_shape=[_VEC_TYPE, _VEC_TYPE, _IDX_TYPE, _LSUM_TYPE],
    )
    decode = pl.pallas_call(
        _decode_kernel,
        grid=_GRID,
        in_specs=[
            _VEC_SPEC, _VEC_SPEC, _VEC_SPEC,
            _full((E_DIM, 128)), _full((1, 128)),
            _full((128, 512)), _full((1, 512)),
            _full((512, IN_DIM)), _full((1, IN_DIM)),
        ],
        out_specs=[_row_block(IN_DIM), _LSUM_SPEC],
        out_shape=[jax.ShapeDtypeStruct((HALF, IN_DIM), jnp.float32), _LSUM_TYPE],
    )

    be = [b.reshape(1, -1) for b in (be0, be1, be2)]
    bd = [b.reshape(1, -1) for b in (bd0, bd1, bd2)]
    halves = []
    for h in range(NCHAINS):
        xh = x[h * HALF:(h + 1) * HALF]
        z, idx = encode(xh, We0, be[0], We1, be[1], We2, be[2], cbT[0])
        halves.append({"r": z, "acc": jnp.zeros((HALF, E_DIM), jnp.float32),
                       "idxs": [idx], "lsums": []})
    for l in range(3):
        for h in halves:
            q = _sc_gather(cb_pad[l], h["idxs"][-1].reshape(-1))[:, :E_DIM]
            h["r"], h["acc"], idx, ls = step(h["r"], q, h["acc"], cbT[l + 1])
            h["idxs"].append(idx)
            h["lsums"].append(ls)
    outs = []
    for h in halves:
        q3 = _sc_gather(cb_pad[3], h["idxs"][-1].reshape(-1))[:, :E_DIM]
        out_h, ls3 = decode(h["r"], q3, h["acc"], Wd0, bd[0], Wd1, bd[1], Wd2, bd[2])
        h["lsums"].append(ls3)
        outs.append(out_h)

    per_level = jnp.stack(
        [sum(h["lsums"][l][0, 0] for h in halves) for l in range(4)]
    ) / (BATCH * E_DIM)
    rq_loss = jnp.mean((1.0 + BETA) * per_level)
    orth_loss = jnp.zeros((), dtype=jnp.float32)
    indices = jnp.concatenate(
        [jnp.concatenate(h["idxs"], axis=1) for h in halves], axis=0)
    out = jnp.concatenate(outs, axis=0)
    return (out, rq_loss, orth_loss, indices)


# speedup vs baseline: 1.4193x; 1.2613x over previous
import functools

import jax
import jax.numpy as jnp
from jax import lax
from jax.experimental import pallas as pl
from jax.experimental.pallas import tpu as pltpu
from jax.experimental.pallas import tpu_sc as plsc

BATCH = 4096
IN_DIM = 768
E_DIM = 32
K = 8192
BETA = 0.25
TILE = 512

_dot = functools.partial(jax.lax.dot_general, preferred_element_type=jnp.float32)
_NC, _NS = 2, 16
_NW = _NC * _NS


def _argmin_idx(r, cbT):
    csq = jnp.sum(cbT * cbT, axis=0, keepdims=True)
    s = csq + _dot(-2.0 * r, cbT, (((1,), (0,)), ((), ())))
    return jnp.argmin(s, axis=1, keepdims=True).astype(jnp.int32)


def _encode_kernel(x_ref, We0_ref, be0_ref, We1_ref, be1_ref, We2_ref, be2_ref,
                   cbT_ref, z_ref, idx_ref):
    h = jnp.maximum(_dot(x_ref[...], We0_ref[...], (((1,), (0,)), ((), ()))) + be0_ref[...], 0.0)
    h = jnp.maximum(_dot(h, We1_ref[...], (((1,), (0,)), ((), ()))) + be1_ref[...], 0.0)
    z = _dot(h, We2_ref[...], (((1,), (0,)), ((), ()))) + be2_ref[...]
    z_ref[...] = z
    idx_ref[...] = _argmin_idx(z, cbT_ref[...])


def _step_kernel(r_ref, q_ref, acc_ref, cbT_ref, rout_ref, accout_ref, idx_ref,
                 lsum_ref):
    @pl.when(pl.program_id(0) == 0)
    def _init():
        lsum_ref[...] = jnp.zeros_like(lsum_ref)

    r = r_ref[...]
    t = q_ref[...] - r
    lsum_ref[...] += jnp.sum(t * t)
    q_st = r + t
    accout_ref[...] = acc_ref[...] + q_st
    rnew = r - q_st
    rout_ref[...] = rnew
    idx_ref[...] = _argmin_idx(rnew, cbT_ref[...])


def _decode_kernel(r_ref, q_ref, acc_ref, Wd0_ref, bd0_ref, Wd1_ref, bd1_ref,
                   Wd2_ref, bd2_ref, out_ref, lsum_ref):
    @pl.when(pl.program_id(0) == 0)
    def _init():
        lsum_ref[...] = jnp.zeros_like(lsum_ref)

    r = r_ref[...]
    t = q_ref[...] - r
    lsum_ref[...] += jnp.sum(t * t)
    xq = acc_ref[...] + (r + t)
    h = jnp.maximum(_dot(xq, Wd0_ref[...], (((1,), (0,)), ((), ()))) + bd0_ref[...], 0.0)
    h = jnp.maximum(_dot(h, Wd1_ref[...], (((1,), (0,)), ((), ()))) + bd1_ref[...], 0.0)
    out_ref[...] = _dot(h, Wd2_ref[...], (((1,), (0,)), ((), ()))) + bd2_ref[...]


NCHAINS = 2
HALF = BATCH // NCHAINS


def _row_block(d1):
    return pl.BlockSpec((TILE, d1), lambda i: (i, 0))


def _full(shape):
    return pl.BlockSpec(shape, lambda i: (0,) * len(shape))


_GRID = (HALF // TILE,)
_IDX_SPEC = pl.BlockSpec((TILE, 1), lambda i: (i, 0))
_IDX_TYPE = jax.ShapeDtypeStruct((HALF, 1), jnp.int32)
_LSUM_SPEC = _full((1, 128))
_LSUM_TYPE = jax.ShapeDtypeStruct((1, 128), jnp.float32)
_VEC_SPEC = pl.BlockSpec((TILE, E_DIM), lambda i: (i, 0))
_VEC_TYPE = jax.ShapeDtypeStruct((HALF, E_DIM), jnp.float32)


def _sc_gather(table, idx):
    nrows = idx.shape[0]
    bpw = nrows // _NW
    width = table.shape[1]
    mesh = plsc.VectorSubcoreMesh(core_axis_name="c", subcore_axis_name="s")

    @functools.partial(
        pl.kernel,
        out_type=jax.ShapeDtypeStruct((nrows, width), jnp.float32),
        mesh=mesh,
        scratch_types=[
            pltpu.VMEM((bpw,), jnp.int32),
            pltpu.VMEM((bpw, width), jnp.float32),
            pltpu.SemaphoreType.DMA,
        ],
    )
    def gather(table_hbm, idx_hbm, out_hbm, idx_v, rows_v, sem):
        wid = lax.axis_index("s") * _NC + lax.axis_index("c")
        base = wid * bpw
        pltpu.sync_copy(idx_hbm.at[pl.ds(base, bpw)], idx_v)
        pltpu.async_copy(table_hbm.at[idx_v], rows_v, sem).wait()
        pltpu.sync_copy(rows_v, out_hbm.at[pl.ds(base, bpw)])

    return gather(table, idx)


def kernel(x, We0, be0, We1, be1, We2, be2, Wd0, bd0, Wd1, bd1, Wd2, bd2, codebooks):
    cbT = jnp.transpose(codebooks, (0, 2, 1))
    cb_pad = jnp.pad(codebooks, ((0, 0), (0, 0), (0, 128 - E_DIM)))

    encode = pl.pallas_call(
        _encode_kernel,
        grid=_GRID,
        in_specs=[
            _row_block(IN_DIM),
            _full((IN_DIM, 512)), _full((1, 512)),
            _full((512, 128)), _full((1, 128)),
            _full((128, E_DIM)), _full((1, E_DIM)),
            _full((E_DIM, K)),
        ],
        out_specs=[_VEC_SPEC, _IDX_SPEC],
        out_shape=[_VEC_TYPE, _IDX_TYPE],
    )
    step = pl.pallas_call(
        _step_kernel,
        grid=_GRID,
        in_specs=[_VEC_SPEC, _VEC_SPEC, _VEC_SPEC, _full((E_DIM, K))],
        out_specs=[_VEC_SPEC, _VEC_SPEC, _IDX_SPEC, _LSUM_SPEC],
        out_shape=[_VEC_TYPE, _VEC_TYPE, _IDX_TYPE, _LSUM_TYPE],
    )
    decode = pl.pallas_call(
        _decode_kernel,
        grid=_GRID,
        in_specs=[
            _VEC_SPEC, _VEC_SPEC, _VEC_SPEC,
            _full((E_DIM, 128)), _full((1, 128)),
            _full((128, 512)), _full((1, 512)),
            _full((512, IN_DIM)), _full((1, IN_DIM)),
        ],
        out_specs=[_row_block(IN_DIM), _LSUM_SPEC],
        out_shape=[jax.ShapeDtypeStruct((HALF, IN_DIM), jnp.float32), _LSUM_TYPE],
    )

    be = [b.reshape(1, -1) for b in (be0, be1, be2)]
    bd = [b.reshape(1, -1) for b in (bd0, bd1, bd2)]
    halves = []
    for h in range(NCHAINS):
        xh = x[h * HALF:(h + 1) * HALF]
        z, idx = encode(xh, We0, be[0], We1, be[1], We2, be[2], cbT[0])
        halves.append({"r": z, "acc": jnp.zeros((HALF, E_DIM), jnp.float32),
                       "idxs": [idx], "lsums": []})
    for l in range(3):
        for h in halves:
            q = _sc_gather(cb_pad[l], h["idxs"][-1].reshape(-1))[:, :E_DIM]
            h["r"], h["acc"], idx, ls = step(h["r"], q, h["acc"], cbT[l + 1])
            h["idxs"].append(idx)
            h["lsums"].append(ls)
    outs = []
    for h in halves:
        q3 = _sc_gather(cb_pad[3], h["idxs"][-1].reshape(-1))[:, :E_DIM]
        out_h, ls3 = decode(h["r"], q3, h["acc"], Wd0, bd[0], Wd1, bd[1], Wd2, bd[2])
        h["lsums"].append(ls3)
        outs.append(out_h)

    per_level = jnp.stack(
        [sum(h["lsums"][l][0, 0] for h in halves) for l in range(4)]
    ) / (BATCH * E_DIM)
    rq_loss = jnp.mean((1.0 + BETA) * per_level)
    orth_loss = jnp.zeros((), dtype=jnp.float32)
    indices = jnp.concatenate(
        [jnp.concatenate(h["idxs"], axis=1) for h in halves], axis=0)
    out = jnp.concatenate(outs, axis=0)
    return (out, rq_loss, orth_loss, indices)
